# Initial kernel scaffold; baseline (speedup 1.0000x reference)
import jax
import jax.numpy as jnp
from jax.experimental import pallas as pl


def kernel(x, edge_index, W1, b1, W2, b2):
    raise NotImplementedError("write your pallas kernel here")


# speedup vs baseline: 28.7833x; 28.7833x over previous
import functools

import jax
import jax.numpy as jnp
from jax import lax
from jax.experimental import pallas as pl
from jax.experimental.pallas import tpu as pltpu
from jax.experimental.pallas import tpu_sc as plsc

_N = 10000
_NP = 10240
_E = 320000
_W = 128
_NC = 2
_NS = 16
_NW = _NC * _NS
_WPW = 80
_WINDP = _NW * _WPW
_EP = _WINDP * _W
_D = 128
_RPT = _NP // _NS

_mesh = plsc.VectorSubcoreMesh(
    core_axis_name="c", subcore_axis_name="s",
    num_cores=_NC, num_subcores=_NS)


def _sc_row_agg_body(xs_hbm, src_hbm, dst_hbm, out_hbm, sidx, didx, rows, acc, sem):
    cid = lax.axis_index("c")
    sid = lax.axis_index("s")
    wid = cid * _NS + sid

    def zb(i, _):
        r = i // 8
        c = (i % 8) * 16
        rows[r, pl.ds(c, 16)] = jnp.zeros((16,), jnp.float32)
        return 0
    lax.fori_loop(0, 1024, zb, 0)
    base = sid * _RPT
    for j in range(_RPT // _W):
        pltpu.sync_copy(rows, acc.at[pl.ds(base + j * _W, _W)])
    plsc.subcore_barrier()

    wstart = wid * _WPW
    pltpu.sync_copy(src_hbm.at[pl.ds(wstart, _WPW)], sidx)
    pltpu.sync_copy(dst_hbm.at[pl.ds(wstart, _WPW)], didx)

    def body(i, _):
        pltpu.async_copy(xs_hbm.at[sidx.at[i]], rows, sem).wait()
        pltpu.sync_copy(rows, acc.at[didx.at[i]], add=True)
        return 0
    lax.fori_loop(0, _WPW, body, 0)
    plsc.subcore_barrier()

    pltpu.sync_copy(acc.at[pl.ds(base, _RPT)], out_hbm.at[cid, pl.ds(base, _RPT)])


_row_agg = pl.kernel(
    _sc_row_agg_body,
    out_type=jax.ShapeDtypeStruct((_NC, _NP, _D), jnp.float32),
    mesh=_mesh,
    scratch_types=[
        pltpu.VMEM((_WPW, _W), jnp.int32),
        pltpu.VMEM((_WPW, _W), jnp.int32),
        pltpu.VMEM((_W, _D), jnp.float32),
        pltpu.VMEM_SHARED((_NP, _D), jnp.float32),
        pltpu.SemaphoreType.DMA,
    ],
)


def _sc_elem_agg_body(vals_hbm, src_hbm, dst_hbm, out_hbm, sidx, didx, upd, acc, sem):
    cid = lax.axis_index("c")
    sid = lax.axis_index("s")
    wid = cid * _NS + sid

    def zb(i, _):
        upd[pl.ds(i * 16, 16)] = jnp.zeros((16,), jnp.float32)
        return 0
    lax.fori_loop(0, _W // 16, zb, 0)
    base = sid * _RPT
    for j in range(_RPT // _W):
        pltpu.sync_copy(upd, acc.at[pl.ds(base + j * _W, _W)])
    plsc.subcore_barrier()

    wstart = wid * _WPW
    pltpu.sync_copy(src_hbm.at[pl.ds(wstart, _WPW)], sidx)
    pltpu.sync_copy(dst_hbm.at[pl.ds(wstart, _WPW)], didx)

    def body(i, _):
        pltpu.async_copy(vals_hbm.at[sidx.at[i]], upd, sem).wait()
        pltpu.sync_copy(upd, acc.at[didx.at[i]], add=True)
        return 0
    lax.fori_loop(0, _WPW, body, 0)
    plsc.subcore_barrier()

    pltpu.sync_copy(acc.at[pl.ds(base, _RPT)], out_hbm.at[cid, pl.ds(base, _RPT)])


_elem_agg = pl.kernel(
    _sc_elem_agg_body,
    out_type=jax.ShapeDtypeStruct((_NC, _NP), jnp.float32),
    mesh=_mesh,
    scratch_types=[
        pltpu.VMEM((_WPW, _W), jnp.int32),
        pltpu.VMEM((_WPW, _W), jnp.int32),
        pltpu.VMEM((_W,), jnp.float32),
        pltpu.VMEM_SHARED((_NP,), jnp.float32),
        pltpu.SemaphoreType.DMA,
    ],
)


_R = 400
_G = _N // _R


def _scale_body(x_ref, d0_ref, d1_ref, xs_ref, dv_ref):
    deg = d0_ref[...] + d1_ref[...] + 1.0
    dinv = lax.rsqrt(deg)
    xs_ref[...] = x_ref[...] * dinv
    dv_ref[...] = dinv


_scale_call = pl.pallas_call(
    _scale_body,
    grid=(_G,),
    in_specs=[
        pl.BlockSpec((_R, _D), lambda i: (i, 0)),
        pl.BlockSpec((_R, 1), lambda i: (i, 0)),
        pl.BlockSpec((_R, 1), lambda i: (i, 0)),
    ],
    out_specs=[
        pl.BlockSpec((_R, _D), lambda i: (i, 0)),
        pl.BlockSpec((_R, 1), lambda i: (i, 0)),
    ],
    out_shape=[
        jax.ShapeDtypeStruct((_N, _D), jnp.float32),
        jax.ShapeDtypeStruct((_N, 1), jnp.float32),
    ],
)


def _mlp_body(a0_ref, a1_ref, xs_ref, dv_ref, w1_ref, b1_ref, w2_ref, zs_ref):
    dinv = dv_ref[...]
    y = (a0_ref[0] + a1_ref[0] + xs_ref[...]) * dinv
    h = jnp.dot(y, w1_ref[...], preferred_element_type=jnp.float32,
                precision=lax.Precision.HIGHEST)
    h = jnp.maximum(h + b1_ref[...], 0.0)
    z = jnp.sum(h * w2_ref[...], axis=1, keepdims=True)
    zs_ref[...] = z * dinv


def _make_mlp(d_hid):
    return pl.pallas_call(
        _mlp_body,
        grid=(_G,),
        in_specs=[
            pl.BlockSpec((1, _R, _D), lambda i: (0, i, 0)),
            pl.BlockSpec((1, _R, _D), lambda i: (1, i, 0)),
            pl.BlockSpec((_R, _D), lambda i: (i, 0)),
            pl.BlockSpec((_R, 1), lambda i: (i, 0)),
            pl.BlockSpec((_D, d_hid), lambda i: (0, 0)),
            pl.BlockSpec((1, d_hid), lambda i: (0, 0)),
            pl.BlockSpec((1, d_hid), lambda i: (0, 0)),
        ],
        out_specs=pl.BlockSpec((_R, 1), lambda i: (i, 0)),
        out_shape=jax.ShapeDtypeStruct((_N, 1), jnp.float32),
    )


def _final_body(z0_ref, z1_ref, zs_ref, dv_ref, b2_ref, out_ref):
    out_ref[...] = (z0_ref[...] + z1_ref[...] + zs_ref[...]) * dv_ref[...] + b2_ref[...]


_final_call = pl.pallas_call(
    _final_body,
    grid=(_G,),
    in_specs=[
        pl.BlockSpec((_R, 1), lambda i: (i, 0)),
        pl.BlockSpec((_R, 1), lambda i: (i, 0)),
        pl.BlockSpec((_R, 1), lambda i: (i, 0)),
        pl.BlockSpec((_R, 1), lambda i: (i, 0)),
        pl.BlockSpec((1, 1), lambda i: (0, 0)),
    ],
    out_specs=pl.BlockSpec((_R, 1), lambda i: (i, 0)),
    out_shape=jax.ShapeDtypeStruct((_N, 1), jnp.float32),
)


def kernel(x, edge_index, W1, b1, W2, b2):
    src = edge_index[0].astype(jnp.int32)
    dst = edge_index[1].astype(jnp.int32)

    pad_n = _EP - _E
    ar = jnp.arange(pad_n, dtype=jnp.int32)
    pad_src = (ar * 37) % _N
    pad_dst = _N + ar % (_NP - _N)
    src2d = jnp.concatenate([src, pad_src]).reshape(_WINDP, _W)
    dst2d = jnp.concatenate([dst, pad_dst]).reshape(_WINDP, _W)

    ones = jnp.ones((_N,), jnp.float32)
    degp = _elem_agg(ones, src2d, dst2d)
    xs, dinv = _scale_call(x, degp[0, :_N, None], degp[1, :_N, None])

    aggp = _row_agg(xs, src2d, dst2d)

    d_hid = W1.shape[1]
    zs = _make_mlp(d_hid)(aggp, aggp, xs, dinv, W1, b1.reshape(1, d_hid),
                          W2.reshape(1, d_hid))

    azp = _elem_agg(zs.reshape(_N), src2d, dst2d)
    out = _final_call(azp[0, :_N, None], azp[1, :_N, None], zs, dinv,
                      b2.reshape(1, 1))
    return out


# speedup vs baseline: 42.3145x; 1.4701x over previous
import functools

import jax
import jax.numpy as jnp
from jax import lax
from jax.experimental import pallas as pl
from jax.experimental.pallas import tpu as pltpu
from jax.experimental.pallas import tpu_sc as plsc

_N = 10000
_NP = 10240
_E = 320000
_W = 128
_NC = 2
_NS = 16
_NW = _NC * _NS
_WPW = 80
_WINDP = _NW * _WPW
_EP = _WINDP * _W
_D = 128
_DH = _D // 2
_WPT = _WINDP // _NS
_RPT = _NP // _NS

_mesh = plsc.VectorSubcoreMesh(
    core_axis_name="c", subcore_axis_name="s",
    num_cores=_NC, num_subcores=_NS)


_CW = 16


def _sc_row_agg_body(xs_hbm, src_hbm, dst_hbm, out_hbm, sidx, didx,
                     r0, r1, acc, gsem, ssem):
    cid = lax.axis_index("c")
    sid = lax.axis_index("s")
    wid = cid * _NS + sid

    def zb(i, _):
        for c in range(_D // 16):
            r0[i, pl.ds(c * 16, 16)] = jnp.zeros((16,), jnp.float32)
        return 0
    lax.fori_loop(0, _W, zb, 0)
    base = sid * _RPT
    for j in range(_RPT // _W):
        pltpu.sync_copy(r0, acc.at[pl.ds(base + j * _W, _W)])
    plsc.subcore_barrier()

    wstart = wid * _WPW

    def gfire(w, buf):
        pltpu.async_copy(xs_hbm.at[sidx.at[w]], buf, gsem)

    def gwait(w, buf):
        pltpu.make_async_copy(xs_hbm.at[sidx.at[w]], buf, gsem).wait()

    def sfire(w, buf):
        pltpu.async_copy(buf, acc.at[didx.at[w]], ssem, add=True)

    def swait(w, buf):
        pltpu.make_async_copy(buf, acc.at[didx.at[w]], ssem).wait()

    for c in range(_WPW // _CW):
        pltpu.sync_copy(src_hbm.at[pl.ds(wstart + c * _CW, _CW)], sidx)
        pltpu.sync_copy(dst_hbm.at[pl.ds(wstart + c * _CW, _CW)], didx)
        gfire(0, r0)

        def step(j, _):
            w = 2 * j
            gwait(w, r0)

            @pl.when(j > 0)
            def _():
                swait(w - 1, r1)

            gfire(w + 1, r1)
            sfire(w, r0)
            gwait(w + 1, r1)
            swait(w, r0)

            @pl.when(j < _CW // 2 - 1)
            def _():
                gfire(w + 2, r0)

            sfire(w + 1, r1)
            return 0

        lax.fori_loop(0, _CW // 2, step, 0)
        swait(_CW - 1, r1)

    plsc.subcore_barrier()
    pltpu.sync_copy(acc.at[pl.ds(base, _RPT)], out_hbm.at[cid, pl.ds(base, _RPT)])


_row_agg = pl.kernel(
    _sc_row_agg_body,
    out_type=jax.ShapeDtypeStruct((_NC, _NP, _D), jnp.float32),
    mesh=_mesh,
    scratch_types=[
        pltpu.VMEM((_CW, _W), jnp.int32),
        pltpu.VMEM((_CW, _W), jnp.int32),
        pltpu.VMEM((_W, _D), jnp.float32),
        pltpu.VMEM((_W, _D), jnp.float32),
        pltpu.VMEM_SHARED((_NP, _D), jnp.float32),
        pltpu.SemaphoreType.DMA,
        pltpu.SemaphoreType.DMA,
    ],
)

_RING = 8


def _zero_acc_slice(zbuf, acc, sid):
    def zb(i, _):
        zbuf[pl.ds(i * 16, 16)] = jnp.zeros((16,), jnp.float32)
        return 0
    lax.fori_loop(0, _RPT // 16, zb, 0)
    pltpu.sync_copy(zbuf, acc.at[pl.ds(sid * _RPT, _RPT)])


def _sc_deg_body(src_hbm, dst_hbm, out_hbm, didx, upd, zbuf, acc, ssem):
    cid = lax.axis_index("c")
    sid = lax.axis_index("s")
    wid = cid * _NS + sid

    _zero_acc_slice(zbuf, acc, sid)
    def ob(i, _):
        upd[pl.ds(i * 16, 16)] = jnp.ones((16,), jnp.float32)
        return 0
    lax.fori_loop(0, _W // 16, ob, 0)
    plsc.subcore_barrier()

    wstart = wid * _WPW
    pltpu.sync_copy(dst_hbm.at[pl.ds(wstart, _WPW)], didx)

    def step(j, _):
        w = j * _RING

        @pl.when(j > 0)
        def _():
            for b in range(_RING):
                pltpu.make_async_copy(
                    upd, acc.at[didx.at[w - _RING + b]], ssem).wait()

        for b in range(_RING):
            pltpu.async_copy(upd, acc.at[didx.at[w + b]], ssem, add=True)
        return 0

    lax.fori_loop(0, _WPW // _RING, step, 0)
    for b in range(_RING):
        pltpu.make_async_copy(upd, acc.at[didx.at[_WPW - _RING + b]], ssem).wait()
    plsc.subcore_barrier()

    base = sid * _RPT
    pltpu.sync_copy(acc.at[pl.ds(base, _RPT)], out_hbm.at[cid, pl.ds(base, _RPT)])


_deg_agg = pl.kernel(
    _sc_deg_body,
    out_type=jax.ShapeDtypeStruct((_NC, _NP), jnp.float32),
    mesh=_mesh,
    scratch_types=[
        pltpu.VMEM((_WPW, _W), jnp.int32),
        pltpu.VMEM((_W,), jnp.float32),
        pltpu.VMEM((_RPT,), jnp.float32),
        pltpu.VMEM_SHARED((_NP,), jnp.float32),
        pltpu.SemaphoreType.DMA,
    ],
)


def _sc_elem_agg_body(vals_hbm, src_hbm, dst_hbm, out_hbm, sidx, didx,
                      upds, zbuf, acc, gsem, ssem):
    cid = lax.axis_index("c")
    sid = lax.axis_index("s")
    wid = cid * _NS + sid

    _zero_acc_slice(zbuf, acc, sid)
    plsc.subcore_barrier()

    wstart = wid * _WPW
    pltpu.sync_copy(src_hbm.at[pl.ds(wstart, _WPW)], sidx)
    pltpu.sync_copy(dst_hbm.at[pl.ds(wstart, _WPW)], didx)

    def gfire(w, b):
        pltpu.async_copy(vals_hbm.at[sidx.at[w]], upds.at[b], gsem)

    def gwait(w, b):
        pltpu.make_async_copy(vals_hbm.at[sidx.at[w]], upds.at[b], gsem).wait()

    def sfire(w, b):
        pltpu.async_copy(upds.at[b], acc.at[didx.at[w]], ssem, add=True)

    def swait(w, b):
        pltpu.make_async_copy(upds.at[b], acc.at[didx.at[w]], ssem).wait()

    def step(t, _):
        w = t * 2 * _RING
        for b in range(_RING):
            gfire(w + b, b)
        for b in range(_RING):
            gwait(w + b, b)

        @pl.when(t > 0)
        def _():
            for b in range(_RING):
                swait(w - _RING + b, _RING + b)

        for b in range(_RING):
            sfire(w + b, b)
        for b in range(_RING):
            gfire(w + _RING + b, _RING + b)
        for b in range(_RING):
            gwait(w + _RING + b, _RING + b)
        for b in range(_RING):
            swait(w + b, b)
        for b in range(_RING):
            sfire(w + _RING + b, _RING + b)
        return 0

    lax.fori_loop(0, _WPW // (2 * _RING), step, 0)
    for b in range(_RING):
        swait(_WPW - _RING + b, _RING + b)
    plsc.subcore_barrier()

    base = sid * _RPT
    pltpu.sync_copy(acc.at[pl.ds(base, _RPT)], out_hbm.at[cid, pl.ds(base, _RPT)])


_elem_agg = pl.kernel(
    _sc_elem_agg_body,
    out_type=jax.ShapeDtypeStruct((_NC, _NP), jnp.float32),
    mesh=_mesh,
    scratch_types=[
        pltpu.VMEM((_WPW, _W), jnp.int32),
        pltpu.VMEM((_WPW, _W), jnp.int32),
        pltpu.VMEM((2 * _RING, _W), jnp.float32),
        pltpu.VMEM((_RPT,), jnp.float32),
        pltpu.VMEM_SHARED((_NP,), jnp.float32),
        pltpu.SemaphoreType.DMA,
        pltpu.SemaphoreType.DMA,
    ],
)


_R = 400
_G = _N // _R


def _scale_body(x_ref, d0_ref, d1_ref, xs_ref, dv_ref):
    deg = d0_ref[...] + d1_ref[...] + 1.0
    dinv = 1.0 / jnp.sqrt(deg)
    xb = x_ref[...].astype(jnp.bfloat16).astype(jnp.float32)
    xs_ref[...] = xb * dinv
    dv_ref[...] = dinv


_scale_call = pl.pallas_call(
    _scale_body,
    grid=(_G,),
    in_specs=[
        pl.BlockSpec((_R, _D), lambda i: (i, 0)),
        pl.BlockSpec((_R, 1), lambda i: (i, 0)),
        pl.BlockSpec((_R, 1), lambda i: (i, 0)),
    ],
    out_specs=[
        pl.BlockSpec((_R, _D), lambda i: (i, 0)),
        pl.BlockSpec((_R, 1), lambda i: (i, 0)),
    ],
    out_shape=[
        jax.ShapeDtypeStruct((_N, _D), jnp.float32),
        jax.ShapeDtypeStruct((_N, 1), jnp.float32),
    ],
)


def _mlp_body(a0_ref, a1_ref, xs_ref, dv_ref, w1_ref, b1_ref, w2_ref, zs_ref):
    dinv = dv_ref[...]
    y = (a0_ref[0] + a1_ref[0] + xs_ref[...]) * dinv
    w1b = w1_ref[...].astype(jnp.bfloat16).astype(jnp.float32)
    h = jnp.dot(y, w1b, preferred_element_type=jnp.float32,
                precision=lax.Precision.HIGHEST)
    h = jnp.maximum(h + b1_ref[...], 0.0)
    hb = h.astype(jnp.bfloat16).astype(jnp.float32)
    w2b = w2_ref[...].astype(jnp.bfloat16).astype(jnp.float32)
    z = jnp.sum(hb * w2b, axis=1, keepdims=True)
    zs_ref[...] = z * dinv


def _make_mlp(d_hid):
    return pl.pallas_call(
        _mlp_body,
        grid=(_G,),
        in_specs=[
            pl.BlockSpec((1, _R, _D), lambda i: (0, i, 0)),
            pl.BlockSpec((1, _R, _D), lambda i: (1, i, 0)),
            pl.BlockSpec((_R, _D), lambda i: (i, 0)),
            pl.BlockSpec((_R, 1), lambda i: (i, 0)),
            pl.BlockSpec((_D, d_hid), lambda i: (0, 0)),
            pl.BlockSpec((1, d_hid), lambda i: (0, 0)),
            pl.BlockSpec((1, d_hid), lambda i: (0, 0)),
        ],
        out_specs=pl.BlockSpec((_R, 1), lambda i: (i, 0)),
        out_shape=jax.ShapeDtypeStruct((_N, 1), jnp.float32),
    )


def _final_body(z0_ref, z1_ref, zs_ref, dv_ref, b2_ref, out_ref):
    out_ref[...] = (z0_ref[...] + z1_ref[...] + zs_ref[...]) * dv_ref[...] + b2_ref[...]


_final_call = pl.pallas_call(
    _final_body,
    grid=(_G,),
    in_specs=[
        pl.BlockSpec((_R, 1), lambda i: (i, 0)),
        pl.BlockSpec((_R, 1), lambda i: (i, 0)),
        pl.BlockSpec((_R, 1), lambda i: (i, 0)),
        pl.BlockSpec((_R, 1), lambda i: (i, 0)),
        pl.BlockSpec((1, 1), lambda i: (0, 0)),
    ],
    out_specs=pl.BlockSpec((_R, 1), lambda i: (i, 0)),
    out_shape=jax.ShapeDtypeStruct((_N, 1), jnp.float32),
)


def kernel(x, edge_index, W1, b1, W2, b2):
    src = edge_index[0].astype(jnp.int32)
    dst = edge_index[1].astype(jnp.int32)

    pad_n = _EP - _E
    ar = jnp.arange(pad_n, dtype=jnp.int32)
    pad_src = (ar * 37) % _N
    pad_dst = _N + ar % (_NP - _N)
    src2d = jnp.concatenate([src, pad_src]).reshape(_WINDP, _W)
    dst2d = jnp.concatenate([dst, pad_dst]).reshape(_WINDP, _W)

    degp = _deg_agg(src2d, dst2d)
    xs, dinv = _scale_call(x, degp[0, :_N, None], degp[1, :_N, None])

    aggp = _row_agg(xs, src2d, dst2d)

    d_hid = W1.shape[1]
    zs = _make_mlp(d_hid)(aggp, aggp, xs, dinv, W1, b1.reshape(1, d_hid),
                          W2.reshape(1, d_hid))

    azp = _elem_agg(zs.reshape(_N), src2d, dst2d)
    out = _final_call(azp[0, :_N, None], azp[1, :_N, None], zs, dinv,
                      b2.reshape(1, 1))
    return out


# speedup vs baseline: 43.1624x; 1.0200x over previous
import functools

import jax
import jax.numpy as jnp
from jax import lax
from jax.experimental import pallas as pl
from jax.experimental.pallas import tpu as pltpu
from jax.experimental.pallas import tpu_sc as plsc

_N = 10000
_NP = 10240
_E = 320000
_W = 128
_NC = 2
_NS = 16
_NW = _NC * _NS
_WPW = 80
_WINDP = _NW * _WPW
_EP = _WINDP * _W
_D = 128
_DH = _D // 2
_WPT = _WINDP // _NS
_RPT = _NP // _NS

_mesh = plsc.VectorSubcoreMesh(
    core_axis_name="c", subcore_axis_name="s",
    num_cores=_NC, num_subcores=_NS)


_CW = 16


def _sc_row_agg_body(xs_hbm, src_hbm, dst_hbm, out_hbm, sidx, didx,
                     r0, r1, acc, gsem, ssem):
    cid = lax.axis_index("c")
    sid = lax.axis_index("s")
    wid = cid * _NS + sid

    def zb(i, _):
        for c in range(_D // 16):
            r0[i, pl.ds(c * 16, 16)] = jnp.zeros((16,), jnp.float32)
        return 0
    lax.fori_loop(0, _W, zb, 0)
    base = sid * _RPT
    for j in range(_RPT // _W):
        pltpu.sync_copy(r0, acc.at[pl.ds(base + j * _W, _W)])
    plsc.subcore_barrier()

    wstart = wid * _WPW

    def gfire(w, buf):
        pltpu.async_copy(xs_hbm.at[sidx.at[w]], buf, gsem)

    def gwait(w, buf):
        pltpu.make_async_copy(xs_hbm.at[sidx.at[w]], buf, gsem).wait()

    def sfire(w, buf):
        pltpu.async_copy(buf, acc.at[didx.at[w]], ssem, add=True)

    def swait(w, buf):
        pltpu.make_async_copy(buf, acc.at[didx.at[w]], ssem).wait()

    pltpu.sync_copy(dst_hbm.at[pl.ds(wstart, _WPW)], didx)

    for c in range(_WPW // _CW):
        pltpu.sync_copy(src_hbm.at[pl.ds(wstart + c * _CW, _CW)], sidx)
        gfire(0, r0)

        def step(j, _):
            w = 2 * j
            g = c * _CW + w
            gwait(w, r0)

            @pl.when(g > 0)
            def _():
                swait(g - 1, r1)

            gfire(w + 1, r1)
            sfire(g, r0)
            gwait(w + 1, r1)
            sfire(g + 1, r1)
            swait(g, r0)

            @pl.when(j < _CW // 2 - 1)
            def _():
                gfire(w + 2, r0)

            return 0

        lax.fori_loop(0, _CW // 2, step, 0)

    swait(_WPW - 1, r1)
    plsc.subcore_barrier()
    pltpu.sync_copy(acc.at[pl.ds(base, _RPT)], out_hbm.at[cid, pl.ds(base, _RPT)])


_row_agg = pl.kernel(
    _sc_row_agg_body,
    out_type=jax.ShapeDtypeStruct((_NC, _NP, _D), jnp.float32),
    mesh=_mesh,
    scratch_types=[
        pltpu.VMEM((_CW, _W), jnp.int32),
        pltpu.VMEM((_WPW, _W), jnp.int32),
        pltpu.VMEM((_W, _D), jnp.float32),
        pltpu.VMEM((_W, _D), jnp.float32),
        pltpu.VMEM_SHARED((_NP, _D), jnp.float32),
        pltpu.SemaphoreType.DMA,
        pltpu.SemaphoreType.DMA,
    ],
)

_RING = 8


def _zero_acc_slice(zbuf, acc, sid):
    def zb(i, _):
        zbuf[pl.ds(i * 16, 16)] = jnp.zeros((16,), jnp.float32)
        return 0
    lax.fori_loop(0, _RPT // 16, zb, 0)
    pltpu.sync_copy(zbuf, acc.at[pl.ds(sid * _RPT, _RPT)])


def _sc_deg_body(src_hbm, dst_hbm, out_hbm, didx, upd, zbuf, acc, ssem):
    cid = lax.axis_index("c")
    sid = lax.axis_index("s")
    wid = cid * _NS + sid

    _zero_acc_slice(zbuf, acc, sid)
    def ob(i, _):
        upd[pl.ds(i * 16, 16)] = jnp.ones((16,), jnp.float32)
        return 0
    lax.fori_loop(0, _W // 16, ob, 0)
    plsc.subcore_barrier()

    wstart = wid * _WPW
    pltpu.sync_copy(dst_hbm.at[pl.ds(wstart, _WPW)], didx)

    def step(j, _):
        w = j * _RING

        @pl.when(j > 0)
        def _():
            for b in range(_RING):
                pltpu.make_async_copy(
                    upd, acc.at[didx.at[w - _RING + b]], ssem).wait()

        for b in range(_RING):
            pltpu.async_copy(upd, acc.at[didx.at[w + b]], ssem, add=True)
        return 0

    lax.fori_loop(0, _WPW // _RING, step, 0)
    for b in range(_RING):
        pltpu.make_async_copy(upd, acc.at[didx.at[_WPW - _RING + b]], ssem).wait()
    plsc.subcore_barrier()

    base = sid * _RPT
    pltpu.sync_copy(acc.at[pl.ds(base, _RPT)], out_hbm.at[cid, pl.ds(base, _RPT)])


_deg_agg = pl.kernel(
    _sc_deg_body,
    out_type=jax.ShapeDtypeStruct((_NC, _NP), jnp.float32),
    mesh=_mesh,
    scratch_types=[
        pltpu.VMEM((_WPW, _W), jnp.int32),
        pltpu.VMEM((_W,), jnp.float32),
        pltpu.VMEM((_RPT,), jnp.float32),
        pltpu.VMEM_SHARED((_NP,), jnp.float32),
        pltpu.SemaphoreType.DMA,
    ],
)


def _sc_elem_agg_body(vals_hbm, src_hbm, dst_hbm, out_hbm, sidx, didx,
                      upds, zbuf, acc, gsem, ssem):
    cid = lax.axis_index("c")
    sid = lax.axis_index("s")
    wid = cid * _NS + sid

    _zero_acc_slice(zbuf, acc, sid)
    plsc.subcore_barrier()

    wstart = wid * _WPW
    pltpu.sync_copy(src_hbm.at[pl.ds(wstart, _WPW)], sidx)
    pltpu.sync_copy(dst_hbm.at[pl.ds(wstart, _WPW)], didx)

    def gfire(w, b):
        pltpu.async_copy(vals_hbm.at[sidx.at[w]], upds.at[b], gsem)

    def gwait(w, b):
        pltpu.make_async_copy(vals_hbm.at[sidx.at[w]], upds.at[b], gsem).wait()

    def sfire(w, b):
        pltpu.async_copy(upds.at[b], acc.at[didx.at[w]], ssem, add=True)

    def swait(w, b):
        pltpu.make_async_copy(upds.at[b], acc.at[didx.at[w]], ssem).wait()

    def step(t, _):
        w = t * 2 * _RING
        for b in range(_RING):
            gfire(w + b, b)
        for b in range(_RING):
            gwait(w + b, b)

        @pl.when(t > 0)
        def _():
            for b in range(_RING):
                swait(w - _RING + b, _RING + b)

        for b in range(_RING):
            sfire(w + b, b)
        for b in range(_RING):
            gfire(w + _RING + b, _RING + b)
        for b in range(_RING):
            gwait(w + _RING + b, _RING + b)
        for b in range(_RING):
            swait(w + b, b)
        for b in range(_RING):
            sfire(w + _RING + b, _RING + b)
        return 0

    lax.fori_loop(0, _WPW // (2 * _RING), step, 0)
    for b in range(_RING):
        swait(_WPW - _RING + b, _RING + b)
    plsc.subcore_barrier()

    base = sid * _RPT
    pltpu.sync_copy(acc.at[pl.ds(base, _RPT)], out_hbm.at[cid, pl.ds(base, _RPT)])


_elem_agg = pl.kernel(
    _sc_elem_agg_body,
    out_type=jax.ShapeDtypeStruct((_NC, _NP), jnp.float32),
    mesh=_mesh,
    scratch_types=[
        pltpu.VMEM((_WPW, _W), jnp.int32),
        pltpu.VMEM((_WPW, _W), jnp.int32),
        pltpu.VMEM((2 * _RING, _W), jnp.float32),
        pltpu.VMEM((_RPT,), jnp.float32),
        pltpu.VMEM_SHARED((_NP,), jnp.float32),
        pltpu.SemaphoreType.DMA,
        pltpu.SemaphoreType.DMA,
    ],
)


_R = 400
_G = _N // _R


def _scale_body(x_ref, d0_ref, d1_ref, xs_ref, dv_ref):
    deg = d0_ref[...] + d1_ref[...] + 1.0
    dinv = 1.0 / jnp.sqrt(deg)
    xb = x_ref[...].astype(jnp.bfloat16).astype(jnp.float32)
    xs_ref[...] = xb * dinv
    dv_ref[...] = dinv


_scale_call = pl.pallas_call(
    _scale_body,
    grid=(_G,),
    in_specs=[
        pl.BlockSpec((_R, _D), lambda i: (i, 0)),
        pl.BlockSpec((_R, 1), lambda i: (i, 0)),
        pl.BlockSpec((_R, 1), lambda i: (i, 0)),
    ],
    out_specs=[
        pl.BlockSpec((_R, _D), lambda i: (i, 0)),
        pl.BlockSpec((_R, 1), lambda i: (i, 0)),
    ],
    out_shape=[
        jax.ShapeDtypeStruct((_N, _D), jnp.float32),
        jax.ShapeDtypeStruct((_N, 1), jnp.float32),
    ],
)


def _mlp_body(a0_ref, a1_ref, xs_ref, dv_ref, w1_ref, b1_ref, w2_ref, zs_ref):
    dinv = dv_ref[...]
    y = (a0_ref[0] + a1_ref[0] + xs_ref[...]) * dinv
    w1b = w1_ref[...].astype(jnp.bfloat16).astype(jnp.float32)
    h = jnp.dot(y, w1b, preferred_element_type=jnp.float32,
                precision=lax.Precision.HIGHEST)
    h = jnp.maximum(h + b1_ref[...], 0.0)
    hb = h.astype(jnp.bfloat16).astype(jnp.float32)
    w2b = w2_ref[...].astype(jnp.bfloat16).astype(jnp.float32)
    z = jnp.sum(hb * w2b, axis=1, keepdims=True)
    zs_ref[...] = z * dinv


def _make_mlp(d_hid):
    return pl.pallas_call(
        _mlp_body,
        grid=(_G,),
        in_specs=[
            pl.BlockSpec((1, _R, _D), lambda i: (0, i, 0)),
            pl.BlockSpec((1, _R, _D), lambda i: (1, i, 0)),
            pl.BlockSpec((_R, _D), lambda i: (i, 0)),
            pl.BlockSpec((_R, 1), lambda i: (i, 0)),
            pl.BlockSpec((_D, d_hid), lambda i: (0, 0)),
            pl.BlockSpec((1, d_hid), lambda i: (0, 0)),
            pl.BlockSpec((1, d_hid), lambda i: (0, 0)),
        ],
        out_specs=pl.BlockSpec((_R, 1), lambda i: (i, 0)),
        out_shape=jax.ShapeDtypeStruct((_NP, 1), jnp.float32),
    )


def _final_body(z0_ref, z1_ref, zs_ref, dv_ref, b2_ref, out_ref):
    out_ref[...] = (z0_ref[...] + z1_ref[...] + zs_ref[...]) * dv_ref[...] + b2_ref[...]


_final_call = pl.pallas_call(
    _final_body,
    grid=(_G,),
    in_specs=[
        pl.BlockSpec((_R, 1), lambda i: (i, 0)),
        pl.BlockSpec((_R, 1), lambda i: (i, 0)),
        pl.BlockSpec((_R, 1), lambda i: (i, 0)),
        pl.BlockSpec((_R, 1), lambda i: (i, 0)),
        pl.BlockSpec((1, 1), lambda i: (0, 0)),
    ],
    out_specs=pl.BlockSpec((_R, 1), lambda i: (i, 0)),
    out_shape=jax.ShapeDtypeStruct((_N, 1), jnp.float32),
)


def kernel(x, edge_index, W1, b1, W2, b2):
    src = edge_index[0].astype(jnp.int32)
    dst = edge_index[1].astype(jnp.int32)

    pad_n = _EP - _E
    ar = jnp.arange(pad_n, dtype=jnp.int32)
    pad_src = (ar * 37) % _N
    pad_dst = _N + ar % (_NP - _N)
    src2d = jnp.concatenate([src, pad_src]).reshape(_WINDP, _W)
    dst2d = jnp.concatenate([dst, pad_dst]).reshape(_WINDP, _W)

    degp = _deg_agg(src2d, dst2d)
    xs, dinv = _scale_call(x, degp[0, :_N, None], degp[1, :_N, None])

    aggp = _row_agg(xs, src2d, dst2d)

    d_hid = W1.shape[1]
    zs = _make_mlp(d_hid)(aggp, aggp, xs, dinv, W1, b1.reshape(1, d_hid),
                          W2.reshape(1, d_hid))

    azp = _elem_agg(zs.reshape(_NP), src2d, dst2d)
    out = _final_call(azp[0, :_N, None], azp[1, :_N, None], zs, dinv,
                      b2.reshape(1, 1))
    return out


# speedup vs baseline: 43.9458x; 1.0181x over previous
import functools

import jax
import jax.numpy as jnp
from jax import lax
from jax.experimental import pallas as pl
from jax.experimental.pallas import tpu as pltpu
from jax.experimental.pallas import tpu_sc as plsc

_N = 10000
_NP = 10240
_E = 320000
_W = 128
_NC = 2
_NS = 16
_NW = _NC * _NS
_WPW = 80
_WINDP = _NW * _WPW
_EP = _WINDP * _W
_D = 128
_DH = _D // 2
_WPT = _WINDP // _NS
_RPT = _NP // _NS

_mesh = plsc.VectorSubcoreMesh(
    core_axis_name="c", subcore_axis_name="s",
    num_cores=_NC, num_subcores=_NS)


_CW = 16


def _sc_row_agg_body(xs_hbm, src_hbm, dst_hbm, out_hbm, sidx, didx,
                     r0, r1, acc, gsem, ssem):
    cid = lax.axis_index("c")
    sid = lax.axis_index("s")
    wid = cid * _NS + sid

    def zb(i, _):
        for c in range(_D // 16):
            r0[i, pl.ds(c * 16, 16)] = jnp.zeros((16,), jnp.float32)
        return 0
    lax.fori_loop(0, _W, zb, 0)
    base = sid * _RPT
    for j in range(_RPT // _W):
        pltpu.sync_copy(r0, acc.at[pl.ds(base + j * _W, _W)])
    plsc.subcore_barrier()

    wstart = wid * _WPW

    def gfire(w, buf):
        pltpu.async_copy(xs_hbm.at[sidx.at[w]], buf, gsem)

    def gwait(w, buf):
        pltpu.make_async_copy(xs_hbm.at[sidx.at[w]], buf, gsem).wait()

    def sfire(w, buf):
        pltpu.async_copy(buf, acc.at[didx.at[w]], ssem, add=True)

    def swait(w, buf):
        pltpu.make_async_copy(buf, acc.at[didx.at[w]], ssem).wait()

    pltpu.sync_copy(dst_hbm.at[pl.ds(wstart, _WPW)], didx)

    for c in range(_WPW // _CW):
        pltpu.sync_copy(src_hbm.at[pl.ds(wstart + c * _CW, _CW)], sidx)
        gfire(0, r0)

        def step(j, _):
            w = 2 * j
            g = c * _CW + w
            gwait(w, r0)

            @pl.when(g > 0)
            def _():
                swait(g - 1, r1)

            gfire(w + 1, r1)
            sfire(g, r0)
            gwait(w + 1, r1)
            sfire(g + 1, r1)
            swait(g, r0)

            @pl.when(j < _CW // 2 - 1)
            def _():
                gfire(w + 2, r0)

            return 0

        lax.fori_loop(0, _CW // 2, step, 0)

    swait(_WPW - 1, r1)
    plsc.subcore_barrier()
    pltpu.sync_copy(acc.at[pl.ds(base, _RPT)], out_hbm.at[cid, pl.ds(base, _RPT)])


_row_agg = pl.kernel(
    _sc_row_agg_body,
    out_type=jax.ShapeDtypeStruct((_NC, _NP, _D), jnp.float32),
    mesh=_mesh,
    scratch_types=[
        pltpu.VMEM((_CW, _W), jnp.int32),
        pltpu.VMEM((_WPW, _W), jnp.int32),
        pltpu.VMEM((_W, _D), jnp.float32),
        pltpu.VMEM((_W, _D), jnp.float32),
        pltpu.VMEM_SHARED((_NP, _D), jnp.float32),
        pltpu.SemaphoreType.DMA,
        pltpu.SemaphoreType.DMA,
    ],
)

_RING = 10


def _zero_acc_slice(zbuf, acc, sid):
    def zb(i, _):
        zbuf[pl.ds(i * 16, 16)] = jnp.zeros((16,), jnp.float32)
        return 0
    lax.fori_loop(0, _RPT // 16, zb, 0)
    pltpu.sync_copy(zbuf, acc.at[pl.ds(sid * _RPT, _RPT)])


def _sc_deg_body(src_hbm, dst_hbm, out_hbm, didx, upd, zbuf, acc, ssem):
    cid = lax.axis_index("c")
    sid = lax.axis_index("s")
    wid = cid * _NS + sid

    _zero_acc_slice(zbuf, acc, sid)
    def ob(i, _):
        upd[pl.ds(i * 16, 16)] = jnp.ones((16,), jnp.float32)
        return 0
    lax.fori_loop(0, _W // 16, ob, 0)
    plsc.subcore_barrier()

    wstart = wid * _WPW
    pltpu.sync_copy(dst_hbm.at[pl.ds(wstart, _WPW)], didx)

    def step(j, _):
        w = j * _RING

        @pl.when(j > 0)
        def _():
            for b in range(_RING):
                pltpu.make_async_copy(
                    upd, acc.at[didx.at[w - _RING + b]], ssem).wait()

        for b in range(_RING):
            pltpu.async_copy(upd, acc.at[didx.at[w + b]], ssem, add=True)
        return 0

    lax.fori_loop(0, _WPW // _RING, step, 0)
    for b in range(_RING):
        pltpu.make_async_copy(upd, acc.at[didx.at[_WPW - _RING + b]], ssem).wait()
    plsc.subcore_barrier()

    base = sid * _RPT
    pltpu.sync_copy(acc.at[pl.ds(base, _RPT)], out_hbm.at[cid, pl.ds(base, _RPT)])


_deg_agg = pl.kernel(
    _sc_deg_body,
    out_type=jax.ShapeDtypeStruct((_NC, _NP), jnp.float32),
    mesh=_mesh,
    scratch_types=[
        pltpu.VMEM((_WPW, _W), jnp.int32),
        pltpu.VMEM((_W,), jnp.float32),
        pltpu.VMEM((_RPT,), jnp.float32),
        pltpu.VMEM_SHARED((_NP,), jnp.float32),
        pltpu.SemaphoreType.DMA,
    ],
)


def _sc_elem_agg_body(vals_hbm, src_hbm, dst_hbm, out_hbm, sidx, didx,
                      upds, zbuf, acc, gsem, ssem):
    cid = lax.axis_index("c")
    sid = lax.axis_index("s")
    wid = cid * _NS + sid

    _zero_acc_slice(zbuf, acc, sid)
    plsc.subcore_barrier()

    wstart = wid * _WPW
    pltpu.sync_copy(src_hbm.at[pl.ds(wstart, _WPW)], sidx)
    pltpu.sync_copy(dst_hbm.at[pl.ds(wstart, _WPW)], didx)

    def gfire(w, b):
        pltpu.async_copy(vals_hbm.at[sidx.at[w]], upds.at[b], gsem)

    def gwait(w, b):
        pltpu.make_async_copy(vals_hbm.at[sidx.at[w]], upds.at[b], gsem).wait()

    def sfire(w, b):
        pltpu.async_copy(upds.at[b], acc.at[didx.at[w]], ssem, add=True)

    def swait(w, b):
        pltpu.make_async_copy(upds.at[b], acc.at[didx.at[w]], ssem).wait()

    def step(t, _):
        w = t * 2 * _RING
        for b in range(_RING):
            gfire(w + b, b)
        for b in range(_RING):
            gwait(w + b, b)

        @pl.when(t > 0)
        def _():
            for b in range(_RING):
                swait(w - _RING + b, _RING + b)

        for b in range(_RING):
            sfire(w + b, b)
        for b in range(_RING):
            gfire(w + _RING + b, _RING + b)
        for b in range(_RING):
            gwait(w + _RING + b, _RING + b)
        for b in range(_RING):
            swait(w + b, b)
        for b in range(_RING):
            sfire(w + _RING + b, _RING + b)
        return 0

    lax.fori_loop(0, _WPW // (2 * _RING), step, 0)
    for b in range(_RING):
        swait(_WPW - _RING + b, _RING + b)
    plsc.subcore_barrier()

    base = sid * _RPT
    pltpu.sync_copy(acc.at[pl.ds(base, _RPT)], out_hbm.at[cid, pl.ds(base, _RPT)])


_elem_agg = pl.kernel(
    _sc_elem_agg_body,
    out_type=jax.ShapeDtypeStruct((_NC, _NP), jnp.float32),
    mesh=_mesh,
    scratch_types=[
        pltpu.VMEM((_WPW, _W), jnp.int32),
        pltpu.VMEM((_WPW, _W), jnp.int32),
        pltpu.VMEM((2 * _RING, _W), jnp.float32),
        pltpu.VMEM((_RPT,), jnp.float32),
        pltpu.VMEM_SHARED((_NP,), jnp.float32),
        pltpu.SemaphoreType.DMA,
        pltpu.SemaphoreType.DMA,
    ],
)


_R = 400
_G = _N // _R


def _scale_body(x_ref, d0_ref, d1_ref, xs_ref, dv_ref):
    deg = d0_ref[...] + d1_ref[...] + 1.0
    dinv = 1.0 / jnp.sqrt(deg)
    xb = x_ref[...].astype(jnp.bfloat16).astype(jnp.float32)
    xs_ref[...] = xb * dinv
    dv_ref[...] = dinv


_scale_call = pl.pallas_call(
    _scale_body,
    grid=(_G,),
    in_specs=[
        pl.BlockSpec((_R, _D), lambda i: (i, 0)),
        pl.BlockSpec((_R, 1), lambda i: (i, 0)),
        pl.BlockSpec((_R, 1), lambda i: (i, 0)),
    ],
    out_specs=[
        pl.BlockSpec((_R, _D), lambda i: (i, 0)),
        pl.BlockSpec((_R, 1), lambda i: (i, 0)),
    ],
    out_shape=[
        jax.ShapeDtypeStruct((_N, _D), jnp.float32),
        jax.ShapeDtypeStruct((_N, 1), jnp.float32),
    ],
)


def _mlp_body(a0_ref, a1_ref, xs_ref, dv_ref, w1_ref, b1_ref, w2_ref, zs_ref):
    dinv = dv_ref[...]
    y = (a0_ref[0] + a1_ref[0] + xs_ref[...]) * dinv
    w1b = w1_ref[...].astype(jnp.bfloat16)
    y_hi = y.astype(jnp.bfloat16)
    y_lo = (y - y_hi.astype(jnp.float32)).astype(jnp.bfloat16)
    h = (jnp.dot(y_hi, w1b, preferred_element_type=jnp.float32)
         + jnp.dot(y_lo, w1b, preferred_element_type=jnp.float32))
    h = jnp.maximum(h + b1_ref[...], 0.0)
    hb = h.astype(jnp.bfloat16).astype(jnp.float32)
    w2b = w2_ref[...].astype(jnp.bfloat16).astype(jnp.float32)
    z = jnp.sum(hb * w2b, axis=1, keepdims=True)
    zs_ref[...] = z * dinv


def _make_mlp(d_hid):
    return pl.pallas_call(
        _mlp_body,
        grid=(_G,),
        in_specs=[
            pl.BlockSpec((1, _R, _D), lambda i: (0, i, 0)),
            pl.BlockSpec((1, _R, _D), lambda i: (1, i, 0)),
            pl.BlockSpec((_R, _D), lambda i: (i, 0)),
            pl.BlockSpec((_R, 1), lambda i: (i, 0)),
            pl.BlockSpec((_D, d_hid), lambda i: (0, 0)),
            pl.BlockSpec((1, d_hid), lambda i: (0, 0)),
            pl.BlockSpec((1, d_hid), lambda i: (0, 0)),
        ],
        out_specs=pl.BlockSpec((_R, 1), lambda i: (i, 0)),
        out_shape=jax.ShapeDtypeStruct((_NP, 1), jnp.float32),
    )


def _final_body(z0_ref, z1_ref, zs_ref, dv_ref, b2_ref, out_ref):
    out_ref[...] = (z0_ref[...] + z1_ref[...] + zs_ref[...]) * dv_ref[...] + b2_ref[...]


_final_call = pl.pallas_call(
    _final_body,
    grid=(_G,),
    in_specs=[
        pl.BlockSpec((_R, 1), lambda i: (i, 0)),
        pl.BlockSpec((_R, 1), lambda i: (i, 0)),
        pl.BlockSpec((_R, 1), lambda i: (i, 0)),
        pl.BlockSpec((_R, 1), lambda i: (i, 0)),
        pl.BlockSpec((1, 1), lambda i: (0, 0)),
    ],
    out_specs=pl.BlockSpec((_R, 1), lambda i: (i, 0)),
    out_shape=jax.ShapeDtypeStruct((_N, 1), jnp.float32),
)


def kernel(x, edge_index, W1, b1, W2, b2):
    src = edge_index[0].astype(jnp.int32)
    dst = edge_index[1].astype(jnp.int32)

    pad_n = _EP - _E
    ar = jnp.arange(pad_n, dtype=jnp.int32)
    pad_src = (ar * 37) % _N
    pad_dst = _N + ar % (_NP - _N)
    src2d = jnp.concatenate([src, pad_src]).reshape(_WINDP, _W)
    dst2d = jnp.concatenate([dst, pad_dst]).reshape(_WINDP, _W)

    degp = _deg_agg(src2d, dst2d)
    xs, dinv = _scale_call(x, degp[0, :_N, None], degp[1, :_N, None])

    aggp = _row_agg(xs, src2d, dst2d)

    d_hid = W1.shape[1]
    zs = _make_mlp(d_hid)(aggp, aggp, xs, dinv, W1, b1.reshape(1, d_hid),
                          W2.reshape(1, d_hid))

    azp = _elem_agg(zs.reshape(_NP), src2d, dst2d)
    out = _final_call(azp[0, :_N, None], azp[1, :_N, None], zs, dinv,
                      b2.reshape(1, 1))
    return out


# speedup vs baseline: 48.6803x; 1.1077x over previous
import functools

import jax
import jax.numpy as jnp
from jax import lax
from jax.experimental import pallas as pl
from jax.experimental.pallas import tpu as pltpu
from jax.experimental.pallas import tpu_sc as plsc

_N = 10000
_NP = 10240
_E = 320000
_W = 128
_NC = 2
_NS = 16
_NW = _NC * _NS
_WPW = 80
_WINDP = _NW * _WPW
_EP = _WINDP * _W
_D = 128
_DH = _D // 2
_WPT = _WINDP // _NS
_RPT = _NP // _NS

_mesh = plsc.VectorSubcoreMesh(
    core_axis_name="c", subcore_axis_name="s",
    num_cores=_NC, num_subcores=_NS)


_CW = 16


def _sc_row_agg_body(xs_hbm, src_hbm, dst_hbm, out_hbm, sidx, didx,
                     r0, r1, acc, gsem, ssem):
    cid = lax.axis_index("c")
    sid = lax.axis_index("s")
    wid = cid * _NS + sid

    def zb(i, _):
        for c in range(_D // 16):
            r0[i, pl.ds(c * 16, 16)] = jnp.zeros((16,), jnp.float32)
        return 0
    lax.fori_loop(0, _W, zb, 0)
    base = sid * _RPT
    for j in range(_RPT // _W):
        pltpu.sync_copy(r0, acc.at[pl.ds(base + j * _W, _W)])
    plsc.subcore_barrier()

    wstart = wid * _WPW

    def gfire(w, buf):
        pltpu.async_copy(xs_hbm.at[sidx.at[w]], buf, gsem)

    def gwait(w, buf):
        pltpu.make_async_copy(xs_hbm.at[sidx.at[w]], buf, gsem).wait()

    def sfire(w, buf):
        pltpu.async_copy(buf, acc.at[didx.at[w]], ssem, add=True)

    def swait(w, buf):
        pltpu.make_async_copy(buf, acc.at[didx.at[w]], ssem).wait()

    pltpu.sync_copy(dst_hbm.at[pl.ds(wstart, _WPW)], didx)

    for c in range(_WPW // _CW):
        pltpu.sync_copy(src_hbm.at[pl.ds(wstart + c * _CW, _CW)], sidx)
        gfire(0, r0)

        def step(j, _):
            w = 2 * j
            g = c * _CW + w
            gwait(w, r0)

            @pl.when(g > 0)
            def _():
                swait(g - 1, r1)

            gfire(w + 1, r1)
            sfire(g, r0)
            gwait(w + 1, r1)
            sfire(g + 1, r1)
            swait(g, r0)

            @pl.when(j < _CW // 2 - 1)
            def _():
                gfire(w + 2, r0)

            return 0

        lax.fori_loop(0, _CW // 2, step, 0)

    swait(_WPW - 1, r1)
    plsc.subcore_barrier()
    pltpu.sync_copy(acc.at[pl.ds(base, _RPT)], out_hbm.at[cid, pl.ds(base, _RPT)])


_row_agg = pl.kernel(
    _sc_row_agg_body,
    out_type=jax.ShapeDtypeStruct((_NC, _NP, _D), jnp.float32),
    mesh=_mesh,
    scratch_types=[
        pltpu.VMEM((_CW, _W), jnp.int32),
        pltpu.VMEM((_WPW, _W), jnp.int32),
        pltpu.VMEM((_W, _D), jnp.float32),
        pltpu.VMEM((_W, _D), jnp.float32),
        pltpu.VMEM_SHARED((_NP, _D), jnp.float32),
        pltpu.SemaphoreType.DMA,
        pltpu.SemaphoreType.DMA,
    ],
)

_RING = 10


def _zero_acc_slice(zbuf, acc, sid):
    def zb(i, _):
        zbuf[pl.ds(i * 16, 16)] = jnp.zeros((16,), jnp.float32)
        return 0
    lax.fori_loop(0, _RPT // 16, zb, 0)
    pltpu.sync_copy(zbuf, acc.at[pl.ds(sid * _RPT, _RPT)])


def _sc_deg_body(src_hbm, dst_hbm, out_hbm, didx, upd, zbuf, acc, ssem):
    cid = lax.axis_index("c")
    sid = lax.axis_index("s")
    wid = cid * _NS + sid

    _zero_acc_slice(zbuf, acc, sid)
    def ob(i, _):
        upd[pl.ds(i * 16, 16)] = jnp.ones((16,), jnp.float32)
        return 0
    lax.fori_loop(0, _W // 16, ob, 0)
    plsc.subcore_barrier()

    wstart = wid * _WPW
    pltpu.sync_copy(dst_hbm.at[pl.ds(wstart, _WPW)], didx)

    def step(j, _):
        w = j * _RING

        @pl.when(j > 0)
        def _():
            for b in range(_RING):
                pltpu.make_async_copy(
                    upd, acc.at[didx.at[w - _RING + b]], ssem).wait()

        for b in range(_RING):
            pltpu.async_copy(upd, acc.at[didx.at[w + b]], ssem, add=True)
        return 0

    lax.fori_loop(0, _WPW // _RING, step, 0)
    for b in range(_RING):
        pltpu.make_async_copy(upd, acc.at[didx.at[_WPW - _RING + b]], ssem).wait()
    plsc.subcore_barrier()

    base = sid * _RPT
    pltpu.sync_copy(acc.at[pl.ds(base, _RPT)], out_hbm.at[cid, pl.ds(base, _RPT)])


_deg_agg = pl.kernel(
    _sc_deg_body,
    out_type=jax.ShapeDtypeStruct((_NC, _NP), jnp.float32),
    mesh=_mesh,
    scratch_types=[
        pltpu.VMEM((_WPW, _W), jnp.int32),
        pltpu.VMEM((_W,), jnp.float32),
        pltpu.VMEM((_RPT,), jnp.float32),
        pltpu.VMEM_SHARED((_NP,), jnp.float32),
        pltpu.SemaphoreType.DMA,
    ],
)


def _sc_elem_agg_body(vals_hbm, src_hbm, dst_hbm, out_hbm, sidx, didx,
                      upds, zbuf, acc, gsem, ssem):
    cid = lax.axis_index("c")
    sid = lax.axis_index("s")
    wid = cid * _NS + sid

    _zero_acc_slice(zbuf, acc, sid)
    plsc.subcore_barrier()

    wstart = wid * _WPW
    pltpu.sync_copy(src_hbm.at[pl.ds(wstart, _WPW)], sidx)
    pltpu.sync_copy(dst_hbm.at[pl.ds(wstart, _WPW)], didx)

    def gfire(w, b):
        pltpu.async_copy(vals_hbm.at[sidx.at[w]], upds.at[b], gsem)

    def gwait(w, b):
        pltpu.make_async_copy(vals_hbm.at[sidx.at[w]], upds.at[b], gsem).wait()

    def sfire(w, b):
        pltpu.async_copy(upds.at[b], acc.at[didx.at[w]], ssem, add=True)

    def swait(w, b):
        pltpu.make_async_copy(upds.at[b], acc.at[didx.at[w]], ssem).wait()

    def step(t, _):
        w = t * 2 * _RING
        for b in range(_RING):
            gfire(w + b, b)
        for b in range(_RING):
            gwait(w + b, b)

        @pl.when(t > 0)
        def _():
            for b in range(_RING):
                swait(w - _RING + b, _RING + b)

        for b in range(_RING):
            sfire(w + b, b)
        for b in range(_RING):
            gfire(w + _RING + b, _RING + b)
        for b in range(_RING):
            gwait(w + _RING + b, _RING + b)
        for b in range(_RING):
            swait(w + b, b)
        for b in range(_RING):
            sfire(w + _RING + b, _RING + b)
        return 0

    lax.fori_loop(0, _WPW // (2 * _RING), step, 0)
    for b in range(_RING):
        swait(_WPW - _RING + b, _RING + b)
    plsc.subcore_barrier()

    base = sid * _RPT
    pltpu.sync_copy(acc.at[pl.ds(base, _RPT)], out_hbm.at[cid, pl.ds(base, _RPT)])


_elem_agg = pl.kernel(
    _sc_elem_agg_body,
    out_type=jax.ShapeDtypeStruct((_NC, _NP), jnp.float32),
    mesh=_mesh,
    scratch_types=[
        pltpu.VMEM((_WPW, _W), jnp.int32),
        pltpu.VMEM((_WPW, _W), jnp.int32),
        pltpu.VMEM((2 * _RING, _W), jnp.float32),
        pltpu.VMEM((_RPT,), jnp.float32),
        pltpu.VMEM_SHARED((_NP,), jnp.float32),
        pltpu.SemaphoreType.DMA,
        pltpu.SemaphoreType.DMA,
    ],
)


_R = 2000
_G = _N // _R


def _scale_body(x_ref, d0_ref, d1_ref, xs_ref, dv_ref):
    deg = d0_ref[...] + d1_ref[...] + 1.0
    dinv = 1.0 / jnp.sqrt(deg)
    xb = x_ref[...].astype(jnp.bfloat16).astype(jnp.float32)
    xs_ref[...] = xb * dinv
    dv_ref[...] = dinv


_scale_call = pl.pallas_call(
    _scale_body,
    grid=(_G,),
    in_specs=[
        pl.BlockSpec((_R, _D), lambda i: (i, 0)),
        pl.BlockSpec((_R, 1), lambda i: (i, 0)),
        pl.BlockSpec((_R, 1), lambda i: (i, 0)),
    ],
    out_specs=[
        pl.BlockSpec((_R, _D), lambda i: (i, 0)),
        pl.BlockSpec((_R, 1), lambda i: (i, 0)),
    ],
    out_shape=[
        jax.ShapeDtypeStruct((_N, _D), jnp.float32),
        jax.ShapeDtypeStruct((_N, 1), jnp.float32),
    ],
)


def _mlp_body(a0_ref, a1_ref, xs_ref, dv_ref, w1_ref, b1_ref, w2_ref, zs_ref):
    dinv = dv_ref[...]
    y = (a0_ref[0] + a1_ref[0] + xs_ref[...]) * dinv
    w1b = w1_ref[...].astype(jnp.bfloat16)
    y_hi = y.astype(jnp.bfloat16)
    y_lo = (y - y_hi.astype(jnp.float32)).astype(jnp.bfloat16)
    h = (jnp.dot(y_hi, w1b, preferred_element_type=jnp.float32)
         + jnp.dot(y_lo, w1b, preferred_element_type=jnp.float32))
    h = jnp.maximum(h + b1_ref[...], 0.0)
    hb = h.astype(jnp.bfloat16).astype(jnp.float32)
    w2b = w2_ref[...].astype(jnp.bfloat16).astype(jnp.float32)
    z = jnp.sum(hb * w2b, axis=1, keepdims=True)
    zs_ref[...] = z * dinv


def _make_mlp(d_hid):
    return pl.pallas_call(
        _mlp_body,
        grid=(_G,),
        in_specs=[
            pl.BlockSpec((1, _R, _D), lambda i: (0, i, 0)),
            pl.BlockSpec((1, _R, _D), lambda i: (1, i, 0)),
            pl.BlockSpec((_R, _D), lambda i: (i, 0)),
            pl.BlockSpec((_R, 1), lambda i: (i, 0)),
            pl.BlockSpec((_D, d_hid), lambda i: (0, 0)),
            pl.BlockSpec((1, d_hid), lambda i: (0, 0)),
            pl.BlockSpec((1, d_hid), lambda i: (0, 0)),
        ],
        out_specs=pl.BlockSpec((_R, 1), lambda i: (i, 0)),
        out_shape=jax.ShapeDtypeStruct((_NP, 1), jnp.float32),
    )


def _final_body(z0_ref, z1_ref, zs_ref, dv_ref, b2_ref, out_ref):
    out_ref[...] = (z0_ref[...] + z1_ref[...] + zs_ref[...]) * dv_ref[...] + b2_ref[...]


_final_call = pl.pallas_call(
    _final_body,
    grid=(_G,),
    in_specs=[
        pl.BlockSpec((_R, 1), lambda i: (i, 0)),
        pl.BlockSpec((_R, 1), lambda i: (i, 0)),
        pl.BlockSpec((_R, 1), lambda i: (i, 0)),
        pl.BlockSpec((_R, 1), lambda i: (i, 0)),
        pl.BlockSpec((1, 1), lambda i: (0, 0)),
    ],
    out_specs=pl.BlockSpec((_R, 1), lambda i: (i, 0)),
    out_shape=jax.ShapeDtypeStruct((_N, 1), jnp.float32),
)


def kernel(x, edge_index, W1, b1, W2, b2):
    src = edge_index[0].astype(jnp.int32)
    dst = edge_index[1].astype(jnp.int32)

    pad_n = _EP - _E
    ar = jnp.arange(pad_n, dtype=jnp.int32)
    pad_src = (ar * 37) % _N
    pad_dst = _N + ar % (_NP - _N)
    src2d = jnp.concatenate([src, pad_src]).reshape(_WINDP, _W)
    dst2d = jnp.concatenate([dst, pad_dst]).reshape(_WINDP, _W)

    degp = _deg_agg(src2d, dst2d)
    xs, dinv = _scale_call(x, degp[0, :_N, None], degp[1, :_N, None])

    aggp = _row_agg(xs, src2d, dst2d)

    d_hid = W1.shape[1]
    zs = _make_mlp(d_hid)(aggp, aggp, xs, dinv, W1, b1.reshape(1, d_hid),
                          W2.reshape(1, d_hid))

    azp = _elem_agg(zs.reshape(_NP), src2d, dst2d)
    out = _final_call(azp[0, :_N, None], azp[1, :_N, None], zs, dinv,
                      b2.reshape(1, 1))
    return out


# speedup vs baseline: 49.3475x; 1.0137x over previous
import functools

import jax
import jax.numpy as jnp
from jax import lax
from jax.experimental import pallas as pl
from jax.experimental.pallas import tpu as pltpu
from jax.experimental.pallas import tpu_sc as plsc

_N = 10000
_NP = 10240
_E = 320000
_W = 128
_NC = 2
_NS = 16
_NW = _NC * _NS
_WPW = 80
_WINDP = _NW * _WPW
_EP = _WINDP * _W
_D = 128
_DH = _D // 2
_WPT = _WINDP // _NS
_RPT = _NP // _NS

_mesh = plsc.VectorSubcoreMesh(
    core_axis_name="c", subcore_axis_name="s",
    num_cores=_NC, num_subcores=_NS)


_CW = 16


def _sc_row_agg_body(xs_hbm, src_hbm, dst_hbm, out_hbm, sidx, didx,
                     r0, r1, acc, gsem, ssem):
    cid = lax.axis_index("c")
    sid = lax.axis_index("s")
    wid = cid * _NS + sid

    def zb(i, _):
        for c in range(_D // 16):
            r0[i, pl.ds(c * 16, 16)] = jnp.zeros((16,), jnp.float32)
        return 0
    lax.fori_loop(0, _W, zb, 0)
    base = sid * _RPT
    for j in range(_RPT // _W):
        pltpu.sync_copy(r0, acc.at[pl.ds(base + j * _W, _W)])
    plsc.subcore_barrier()

    wstart = wid * _WPW

    def gfire(w, buf):
        pltpu.async_copy(xs_hbm.at[sidx.at[w]], buf, gsem)

    def gwait(w, buf):
        pltpu.make_async_copy(xs_hbm.at[sidx.at[w]], buf, gsem).wait()

    def sfire(w, buf):
        pltpu.async_copy(buf, acc.at[didx.at[w]], ssem, add=True)

    def swait(w, buf):
        pltpu.make_async_copy(buf, acc.at[didx.at[w]], ssem).wait()

    pltpu.sync_copy(dst_hbm.at[pl.ds(wstart, _WPW)], didx)

    for c in range(_WPW // _CW):
        pltpu.sync_copy(src_hbm.at[pl.ds(wstart + c * _CW, _CW)], sidx)
        gfire(0, r0)

        def step(j, _):
            w = 2 * j
            g = c * _CW + w
            gwait(w, r0)

            @pl.when(g > 0)
            def _():
                swait(g - 1, r1)

            gfire(w + 1, r1)
            sfire(g, r0)
            gwait(w + 1, r1)
            sfire(g + 1, r1)
            swait(g, r0)

            @pl.when(j < _CW // 2 - 1)
            def _():
                gfire(w + 2, r0)

            return 0

        lax.fori_loop(0, _CW // 2, step, 0)

    swait(_WPW - 1, r1)
    plsc.subcore_barrier()
    pltpu.sync_copy(acc.at[pl.ds(base, _RPT)], out_hbm.at[cid, pl.ds(base, _RPT)])


_row_agg = pl.kernel(
    _sc_row_agg_body,
    out_type=jax.ShapeDtypeStruct((_NC, _NP, _D), jnp.float32),
    mesh=_mesh,
    scratch_types=[
        pltpu.VMEM((_CW, _W), jnp.int32),
        pltpu.VMEM((_WPW, _W), jnp.int32),
        pltpu.VMEM((_W, _D), jnp.float32),
        pltpu.VMEM((_W, _D), jnp.float32),
        pltpu.VMEM_SHARED((_NP, _D), jnp.float32),
        pltpu.SemaphoreType.DMA,
        pltpu.SemaphoreType.DMA,
    ],
)

_RING = 10


def _zero_acc_slice(zbuf, acc, sid):
    def zb(i, _):
        zbuf[pl.ds(i * 16, 16)] = jnp.zeros((16,), jnp.float32)
        return 0
    lax.fori_loop(0, _RPT // 16, zb, 0)
    pltpu.sync_copy(zbuf, acc.at[pl.ds(sid * _RPT, _RPT)])


def _sc_deg_body(src_hbm, dst_hbm, out_hbm, didx, upd, zbuf, acc, ssem):
    cid = lax.axis_index("c")
    sid = lax.axis_index("s")
    wid = cid * _NS + sid

    _zero_acc_slice(zbuf, acc, sid)
    def ob(i, _):
        upd[pl.ds(i * 16, 16)] = jnp.ones((16,), jnp.float32)
        return 0
    lax.fori_loop(0, _W // 16, ob, 0)
    plsc.subcore_barrier()

    wstart = wid * _WPW
    pltpu.sync_copy(dst_hbm.at[pl.ds(wstart, _WPW)], didx)

    def step(j, _):
        w = j * _RING

        @pl.when(j > 0)
        def _():
            for b in range(_RING):
                pltpu.make_async_copy(
                    upd, acc.at[didx.at[w - _RING + b]], ssem).wait()

        for b in range(_RING):
            pltpu.async_copy(upd, acc.at[didx.at[w + b]], ssem, add=True)
        return 0

    lax.fori_loop(0, _WPW // _RING, step, 0)
    for b in range(_RING):
        pltpu.make_async_copy(upd, acc.at[didx.at[_WPW - _RING + b]], ssem).wait()
    plsc.subcore_barrier()

    base = sid * _RPT
    pltpu.sync_copy(acc.at[pl.ds(base, _RPT)], out_hbm.at[cid, pl.ds(base, _RPT)])


_deg_agg = pl.kernel(
    _sc_deg_body,
    out_type=jax.ShapeDtypeStruct((_NC, _NP), jnp.float32),
    mesh=_mesh,
    scratch_types=[
        pltpu.VMEM((_WPW, _W), jnp.int32),
        pltpu.VMEM((_W,), jnp.float32),
        pltpu.VMEM((_RPT,), jnp.float32),
        pltpu.VMEM_SHARED((_NP,), jnp.float32),
        pltpu.SemaphoreType.DMA,
    ],
)


def _sc_elem_agg_body(vals_hbm, src_hbm, dst_hbm, out_hbm, sidx, didx,
                      upds, zbuf, acc, gsem, ssem):
    cid = lax.axis_index("c")
    sid = lax.axis_index("s")
    wid = cid * _NS + sid

    _zero_acc_slice(zbuf, acc, sid)
    plsc.subcore_barrier()

    wstart = wid * _WPW
    pltpu.sync_copy(src_hbm.at[pl.ds(wstart, _WPW)], sidx)
    pltpu.sync_copy(dst_hbm.at[pl.ds(wstart, _WPW)], didx)

    def gfire(w, b):
        pltpu.async_copy(vals_hbm.at[sidx.at[w]], upds.at[b], gsem)

    def gwait(w, b):
        pltpu.make_async_copy(vals_hbm.at[sidx.at[w]], upds.at[b], gsem).wait()

    def sfire(w, b):
        pltpu.async_copy(upds.at[b], acc.at[didx.at[w]], ssem, add=True)

    def swait(w, b):
        pltpu.make_async_copy(upds.at[b], acc.at[didx.at[w]], ssem).wait()

    def step(t, _):
        w = t * 2 * _RING
        for b in range(_RING):
            gfire(w + b, b)
        for b in range(_RING):
            gwait(w + b, b)

        @pl.when(t > 0)
        def _():
            for b in range(_RING):
                swait(w - _RING + b, _RING + b)

        for b in range(_RING):
            sfire(w + b, b)
        for b in range(_RING):
            gfire(w + _RING + b, _RING + b)
        for b in range(_RING):
            gwait(w + _RING + b, _RING + b)
        for b in range(_RING):
            swait(w + b, b)
        for b in range(_RING):
            sfire(w + _RING + b, _RING + b)
        return 0

    lax.fori_loop(0, _WPW // (2 * _RING), step, 0)
    for b in range(_RING):
        swait(_WPW - _RING + b, _RING + b)
    plsc.subcore_barrier()

    base = sid * _RPT
    pltpu.sync_copy(acc.at[pl.ds(base, _RPT)], out_hbm.at[cid, pl.ds(base, _RPT)])


_elem_agg = pl.kernel(
    _sc_elem_agg_body,
    out_type=jax.ShapeDtypeStruct((_NC, _NP), jnp.float32),
    mesh=_mesh,
    scratch_types=[
        pltpu.VMEM((_WPW, _W), jnp.int32),
        pltpu.VMEM((_WPW, _W), jnp.int32),
        pltpu.VMEM((2 * _RING, _W), jnp.float32),
        pltpu.VMEM((_RPT,), jnp.float32),
        pltpu.VMEM_SHARED((_NP,), jnp.float32),
        pltpu.SemaphoreType.DMA,
        pltpu.SemaphoreType.DMA,
    ],
)


_R = 5000
_G = _N // _R


def _scale_body(x_ref, d0_ref, d1_ref, xs_ref, dv_ref):
    deg = d0_ref[...] + d1_ref[...] + 1.0
    dinv = 1.0 / jnp.sqrt(deg)
    xb = x_ref[...].astype(jnp.bfloat16).astype(jnp.float32)
    xs_ref[...] = xb * dinv
    dv_ref[...] = dinv


_scale_call = pl.pallas_call(
    _scale_body,
    grid=(_G,),
    in_specs=[
        pl.BlockSpec((_R, _D), lambda i: (i, 0)),
        pl.BlockSpec((_R, 1), lambda i: (i, 0)),
        pl.BlockSpec((_R, 1), lambda i: (i, 0)),
    ],
    out_specs=[
        pl.BlockSpec((_R, _D), lambda i: (i, 0)),
        pl.BlockSpec((_R, 1), lambda i: (i, 0)),
    ],
    out_shape=[
        jax.ShapeDtypeStruct((_N, _D), jnp.float32),
        jax.ShapeDtypeStruct((_N, 1), jnp.float32),
    ],
)


def _mlp_body(a0_ref, a1_ref, xs_ref, dv_ref, w1_ref, b1_ref, w2_ref, zs_ref):
    dinv = dv_ref[...]
    y = (a0_ref[0] + a1_ref[0] + xs_ref[...]) * dinv
    w1b = w1_ref[...].astype(jnp.bfloat16)
    y_hi = y.astype(jnp.bfloat16)
    y_lo = (y - y_hi.astype(jnp.float32)).astype(jnp.bfloat16)
    h = (jnp.dot(y_hi, w1b, preferred_element_type=jnp.float32)
         + jnp.dot(y_lo, w1b, preferred_element_type=jnp.float32))
    h = jnp.maximum(h + b1_ref[...], 0.0)
    hb = h.astype(jnp.bfloat16).astype(jnp.float32)
    w2b = w2_ref[...].astype(jnp.bfloat16).astype(jnp.float32)
    z = jnp.sum(hb * w2b, axis=1, keepdims=True)
    zs_ref[...] = z * dinv


def _make_mlp(d_hid):
    return pl.pallas_call(
        _mlp_body,
        grid=(_G,),
        in_specs=[
            pl.BlockSpec((1, _R, _D), lambda i: (0, i, 0)),
            pl.BlockSpec((1, _R, _D), lambda i: (1, i, 0)),
            pl.BlockSpec((_R, _D), lambda i: (i, 0)),
            pl.BlockSpec((_R, 1), lambda i: (i, 0)),
            pl.BlockSpec((_D, d_hid), lambda i: (0, 0)),
            pl.BlockSpec((1, d_hid), lambda i: (0, 0)),
            pl.BlockSpec((1, d_hid), lambda i: (0, 0)),
        ],
        out_specs=pl.BlockSpec((_R, 1), lambda i: (i, 0)),
        out_shape=jax.ShapeDtypeStruct((_NP, 1), jnp.float32),
    )


def _final_body(z0_ref, z1_ref, zs_ref, dv_ref, b2_ref, out_ref):
    out_ref[...] = (z0_ref[...] + z1_ref[...] + zs_ref[...]) * dv_ref[...] + b2_ref[...]


_final_call = pl.pallas_call(
    _final_body,
    grid=(_G,),
    in_specs=[
        pl.BlockSpec((_R, 1), lambda i: (i, 0)),
        pl.BlockSpec((_R, 1), lambda i: (i, 0)),
        pl.BlockSpec((_R, 1), lambda i: (i, 0)),
        pl.BlockSpec((_R, 1), lambda i: (i, 0)),
        pl.BlockSpec((1, 1), lambda i: (0, 0)),
    ],
    out_specs=pl.BlockSpec((_R, 1), lambda i: (i, 0)),
    out_shape=jax.ShapeDtypeStruct((_N, 1), jnp.float32),
)


def kernel(x, edge_index, W1, b1, W2, b2):
    src = edge_index[0].astype(jnp.int32)
    dst = edge_index[1].astype(jnp.int32)

    pad_n = _EP - _E
    ar = jnp.arange(pad_n, dtype=jnp.int32)
    pad_src = (ar * 37) % _N
    pad_dst = _N + ar % (_NP - _N)
    src2d = jnp.concatenate([src, pad_src]).reshape(_WINDP, _W)
    dst2d = jnp.concatenate([dst, pad_dst]).reshape(_WINDP, _W)

    degp = _deg_agg(src2d, dst2d)
    xs, dinv = _scale_call(x, degp[0, :_N, None], degp[1, :_N, None])

    aggp = _row_agg(xs, src2d, dst2d)

    d_hid = W1.shape[1]
    zs = _make_mlp(d_hid)(aggp, aggp, xs, dinv, W1, b1.reshape(1, d_hid),
                          W2.reshape(1, d_hid))

    azp = _elem_agg(zs.reshape(_NP), src2d, dst2d)
    out = _final_call(azp[0, :_N, None], azp[1, :_N, None], zs, dinv,
                      b2.reshape(1, 1))
    return out
